# baseline (speedup 1.0000x reference)
import functools

import jax
import jax.numpy as jnp
from jax.experimental import pallas as pl
from jax.experimental.pallas import tpu as pltpu

_B, _A, _S, _C = 32, 5, 52, 80
_CELLS = _S * _S
_LCOORD = 5.0
_LNOOBJ = 0.5


def _body(anchors_ref, p_ref, t_ref, out_ref):
    i = pl.program_id(0)
    a_idx = i % _A
    aw = anchors_ref[a_idx, 0]
    ah = anchors_ref[a_idx, 1]

    p = p_ref[0]
    t = t_ref[0]

    col = jax.lax.broadcasted_iota(jnp.int32, (1, 5 + _C), 1)
    is_cls = col >= 5

    t0 = t[:, 0:1]
    obj = (t0 == 1.0).astype(jnp.float32)
    noobj = (t0 == 0.0).astype(jnp.float32)

    ex = jnp.exp(p)
    sg = ex / (1.0 + ex)

    zero = jnp.zeros_like(p)
    s = jnp.sum(jnp.where(is_cls, ex, zero), axis=1, keepdims=True)
    ll = jnp.sum(jnp.where(is_cls, t * p, zero), axis=1, keepdims=True)
    ce = jnp.log(s) - ll

    px = sg[:, 1:2]
    py = sg[:, 2:3]
    pw = ex[:, 3:4] * aw
    ph = ex[:, 4:5] * ah
    tx = t[:, 1:2]
    ty = t[:, 2:3]
    tw = t[:, 3:4]
    th = t[:, 4:5]

    coords = (px - tx) ** 2 + (py - ty) ** 2 + (pw - tw) ** 2 + (ph - th) ** 2

    hw_p, hh_p = pw * 0.5, ph * 0.5
    hw_t, hh_t = tw * 0.5, th * 0.5
    ix = jnp.maximum(
        jnp.minimum(px + hw_p, tx + hw_t) - jnp.maximum(px - hw_p, tx - hw_t), 0.0
    )
    iy = jnp.maximum(
        jnp.minimum(py + hh_p, ty + hh_t) - jnp.maximum(py - hh_p, ty - hh_t), 0.0
    )
    inter = ix * iy
    union = pw * ph + tw * th - inter + 1e-6
    iou = inter / union

    sg0 = sg[:, 0:1]
    obj_term = obj * (sg0 - iou) ** 2
    noobj_term = noobj * (sg0 - t0) ** 2

    block_total = jnp.sum(
        obj * (_LCOORD * coords + ce) + obj_term + _LNOOBJ * noobj_term
    )

    @pl.when(i == 0)
    def _init():
        out_ref[0, 0] = 0.0

    out_ref[0, 0] += block_total


def _yolo_loss(preds, target, anchors):
    p = preds.reshape(_B * _A, _CELLS, 5 + _C)
    t = target.reshape(_B * _A, _CELLS, 5 + _C)
    out = pl.pallas_call(
        _body,
        grid=(_B * _A,),
        in_specs=[
            pl.BlockSpec(memory_space=pltpu.SMEM),
            pl.BlockSpec((1, _CELLS, 5 + _C), lambda i: (i, 0, 0)),
            pl.BlockSpec((1, _CELLS, 5 + _C), lambda i: (i, 0, 0)),
        ],
        out_specs=pl.BlockSpec(memory_space=pltpu.SMEM),
        out_shape=jax.ShapeDtypeStruct((1, 1), jnp.float32),
    )(anchors, p, t)
    return out[0, 0]


def kernel(preds, target, anchors):
    return _yolo_loss(preds, target, anchors)
